# Initial kernel scaffold; baseline (speedup 1.0000x reference)
import jax
import jax.numpy as jnp
from jax.experimental import pallas as pl


def kernel(x, positional_params):
    raise NotImplementedError("write your pallas kernel here")


# speedup vs baseline: 6.7723x; 6.7723x over previous
import functools

import jax
import jax.numpy as jnp
from jax import lax
from jax.experimental import pallas as pl
from jax.experimental.pallas import tpu as pltpu
from jax.experimental.pallas import tpu_sc as plsc

_HIDDEN = 64
_MAX_LEN = 2048
_TABLE_ROWS = 2 * _MAX_LEN - 1


def _make_sc_kernel(S: int, D: int, T: int):
    info = plsc.get_sparse_core_info()
    num_cores, num_subcores = info.num_cores, info.num_subcores
    num_workers = num_cores * num_subcores
    rows_per_worker = S // num_workers

    mesh = plsc.VectorSubcoreMesh(core_axis_name="c", subcore_axis_name="s")

    @functools.partial(
        pl.kernel,
        mesh=mesh,
        out_type=jax.ShapeDtypeStruct((S, S, D), jnp.float32),
        scratch_types=[pltpu.VMEM_SHARED((T, D), jnp.float32)],
    )
    def sc_kernel(table_hbm, out_hbm, spmem_table):
        c = lax.axis_index("c")
        s = lax.axis_index("s")

        @pl.when(s == 0)
        def _stage():
            pltpu.sync_copy(table_hbm, spmem_table)

        plsc.subcore_barrier()

        wid = c * num_subcores + s
        base = wid * rows_per_worker

        @pl.loop(0, rows_per_worker)
        def _row(k):
            i = base + k
            src = spmem_table.at[pl.ds((S - 1) - i, S)]
            pltpu.sync_copy(src, out_hbm.at[i])

    return sc_kernel


_sc_kernel = _make_sc_kernel(_MAX_LEN, _HIDDEN, _TABLE_ROWS)


def kernel(x, positional_params):
    del x
    return _sc_kernel(positional_params)


# speedup vs baseline: 6.7999x; 1.0041x over previous
import functools

import jax
import jax.numpy as jnp
from jax import lax
from jax.experimental import pallas as pl
from jax.experimental.pallas import tpu as pltpu
from jax.experimental.pallas import tpu_sc as plsc

_HIDDEN = 64
_MAX_LEN = 2048
_TABLE_ROWS = 2 * _MAX_LEN - 1


def _make_sc_kernel(S: int, D: int, T: int):
    info = plsc.get_sparse_core_info()
    num_cores, num_subcores = info.num_cores, info.num_subcores
    num_workers = num_cores * num_subcores
    rows_per_worker = S // num_workers

    mesh = plsc.VectorSubcoreMesh(core_axis_name="c", subcore_axis_name="s")

    depth = 8

    @functools.partial(
        pl.kernel,
        mesh=mesh,
        out_type=jax.ShapeDtypeStruct((S, S, D), jnp.float32),
        scratch_types=[
            pltpu.VMEM_SHARED((T, D), jnp.float32),
            pltpu.SemaphoreType.DMA,
        ],
    )
    def sc_kernel(table_hbm, out_hbm, spmem_table, sem):
        c = lax.axis_index("c")
        s = lax.axis_index("s")

        @pl.when(s == 0)
        def _stage():
            pltpu.sync_copy(table_hbm, spmem_table)

        plsc.subcore_barrier()

        wid = c * num_subcores + s
        base = wid * rows_per_worker

        def copy_descr(i):
            src = spmem_table.at[pl.ds((S - 1) - i, S)]
            return pltpu.make_async_copy(src, out_hbm.at[i], sem)

        for t in range(depth):
            copy_descr(base + t).start()

        @pl.loop(0, rows_per_worker - depth)
        def _steady(k):
            copy_descr(base + k).wait()
            copy_descr(base + k + depth).start()

        @pl.loop(0, depth)
        def _drain(k):
            copy_descr(base + rows_per_worker - depth + k).wait()

    return sc_kernel


_sc_kernel = _make_sc_kernel(_MAX_LEN, _HIDDEN, _TABLE_ROWS)


def kernel(x, positional_params):
    del x
    return _sc_kernel(positional_params)
